# Initial kernel scaffold; baseline (speedup 1.0000x reference)
import functools

import jax
import jax.numpy as jnp
from jax import lax
from jax.experimental import pallas as pl
from jax.experimental.pallas import tpu as pltpu
from jax.experimental.pallas import tpu_sc as plsc

N_NODES = 10000
D_FEAT = 256
DH = 128
N_EDGES = 160000
CHK = 128
E_PAD = 163840
E_ROWS = E_PAD // CHK
NC = 2
NS = 16
LANES = 16
H_PAD = 10240
SENT = N_NODES


def _deg_body(dst2d, deg_parts, hist_sh, idx_v, ones_v, zero_v):
    c = lax.axis_index("c")
    s = lax.axis_index("s")

    def fill(i, _):
        ones_v[i, :] = jnp.full((LANES,), 1.0, jnp.float32)
        zero_v[i, :] = jnp.zeros((LANES,), jnp.float32)
        return 0

    lax.fori_loop(0, CHK, fill, 0)

    stripe = H_PAD // NS
    for k in range(stripe // CHK):
        pltpu.sync_copy(zero_v, hist_sh.at[pl.ds(s * stripe + k * CHK, CHK)])
    plsc.subcore_barrier()

    rows_per_tile = E_ROWS // (NC * NS)
    base = c * (E_ROWS // NC) + s * rows_per_tile
    pltpu.sync_copy(dst2d.at[pl.ds(base, rows_per_tile)], idx_v)
    for j in range(rows_per_tile):
        pltpu.sync_copy(ones_v, hist_sh.at[idx_v.at[j]], add=True)
    plsc.subcore_barrier()

    pltpu.sync_copy(hist_sh.at[pl.ds(s * stripe, stripe)],
                    deg_parts.at[c, pl.ds(s * stripe, stripe)])


def _run_deg(dst2d):
    mesh = plsc.VectorSubcoreMesh(core_axis_name="c", subcore_axis_name="s")
    return pl.kernel(
        _deg_body,
        out_type=jax.ShapeDtypeStruct((NC, H_PAD, LANES), jnp.float32),
        mesh=mesh,
        scratch_types=[
            pltpu.VMEM_SHARED((H_PAD, LANES), jnp.float32),
            pltpu.VMEM((E_ROWS // (NC * NS), CHK), jnp.int32),
            pltpu.VMEM((CHK, LANES), jnp.float32),
            pltpu.VMEM((CHK, LANES), jnp.float32),
        ],
    )(dst2d)


def _scatter_body(y3, src2d, dst2d, acc3, acc_sh, sidx_v, didx_v, rows_v,
                  zero_v, sem):
    c = lax.axis_index("c")
    s = lax.axis_index("s")

    def fill(i, _):
        for k in range(DH // LANES):
            zero_v[i, pl.ds(k * LANES, LANES)] = jnp.zeros((LANES,),
                                                           jnp.float32)
        return 0

    lax.fori_loop(0, CHK, fill, 0)

    stripe = H_PAD // NS
    for k in range(stripe // CHK):
        pltpu.sync_copy(zero_v, acc_sh.at[pl.ds(s * stripe + k * CHK, CHK)])
    plsc.subcore_barrier()

    rows_per_tile = E_ROWS // NS
    base = s * rows_per_tile
    pltpu.sync_copy(src2d.at[pl.ds(base, rows_per_tile)], sidx_v)
    pltpu.sync_copy(dst2d.at[pl.ds(base, rows_per_tile)], didx_v)

    def step(j, _):
        pltpu.async_copy(y3.at[c].at[sidx_v.at[j]], rows_v, sem).wait()
        pltpu.sync_copy(rows_v, acc_sh.at[didx_v.at[j]], add=True)
        return 0

    lax.fori_loop(0, rows_per_tile, step, 0)
    plsc.subcore_barrier()

    out_stripe = N_NODES // NS
    pltpu.sync_copy(acc_sh.at[pl.ds(s * out_stripe, out_stripe)],
                    acc3.at[c, pl.ds(s * out_stripe, out_stripe)])


def _run_scatter(y3, src2d, dst2d):
    mesh = plsc.VectorSubcoreMesh(core_axis_name="c", subcore_axis_name="s")
    return pl.kernel(
        _scatter_body,
        out_type=jax.ShapeDtypeStruct((NC, N_NODES, DH), jnp.float32),
        mesh=mesh,
        scratch_types=[
            pltpu.VMEM_SHARED((H_PAD, DH), jnp.float32),
            pltpu.VMEM((E_ROWS // NS, CHK), jnp.int32),
            pltpu.VMEM((E_ROWS // NS, CHK), jnp.int32),
            pltpu.VMEM((CHK, DH), jnp.float32),
            pltpu.VMEM((CHK, DH), jnp.float32),
            pltpu.SemaphoreType.DMA,
        ],
    )(y3, src2d, dst2d)


_RB = 500


def _mm_body(x_ref, w_ref, dp_ref, y3_ref, dinv_ref):
    deg = dp_ref[0, :, 0:1] + dp_ref[1, :, 0:1] + 1.0
    dinv = lax.rsqrt(deg)
    xw = jnp.dot(x_ref[...], w_ref[...], preferred_element_type=jnp.float32)
    y3_ref[0, :, :] = xw * dinv
    dinv_ref[...] = dinv


def _run_mm(x, W, deg_parts):
    grid = (N_NODES // _RB, NC)
    return pl.pallas_call(
        _mm_body,
        grid=grid,
        in_specs=[
            pl.BlockSpec((_RB, D_FEAT), lambda i, c: (i, 0)),
            pl.BlockSpec((D_FEAT, DH), lambda i, c: (0, c)),
            pl.BlockSpec((NC, _RB, LANES), lambda i, c: (0, i, 0)),
        ],
        out_specs=[
            pl.BlockSpec((1, _RB, DH), lambda i, c: (c, i, 0)),
            pl.BlockSpec((_RB, 1), lambda i, c: (i, 0)),
        ],
        out_shape=[
            jax.ShapeDtypeStruct((NC, N_NODES, DH), jnp.float32),
            jax.ShapeDtypeStruct((N_NODES, 1), jnp.float32),
        ],
    )(x, W, deg_parts)


def _ep_body(acc_ref, y_ref, dinv_ref, b_ref, o_ref):
    o_ref[...] = jnp.maximum(
        dinv_ref[...] * (acc_ref[0] + y_ref[0]) + b_ref[...], 0.0)


def _run_epilogue(acc3, y3, dinv, b2):
    grid = (N_NODES // _RB, NC)
    return pl.pallas_call(
        _ep_body,
        grid=grid,
        in_specs=[
            pl.BlockSpec((1, _RB, DH), lambda i, c: (c, i, 0)),
            pl.BlockSpec((1, _RB, DH), lambda i, c: (c, i, 0)),
            pl.BlockSpec((_RB, 1), lambda i, c: (i, 0)),
            pl.BlockSpec((1, DH), lambda i, c: (c, 0)),
        ],
        out_specs=pl.BlockSpec((_RB, DH), lambda i, c: (i, c)),
        out_shape=jax.ShapeDtypeStruct((N_NODES, D_FEAT), jnp.float32),
    )(acc3, y3, dinv, b2)


@jax.jit
def kernel(x, edge_index, W, b):
    ei = edge_index.astype(jnp.int32)
    pad = E_PAD - N_EDGES
    src2d = jnp.concatenate(
        [ei[0], jnp.zeros((pad,), jnp.int32)]).reshape(E_ROWS, CHK)
    dst2d = jnp.concatenate(
        [ei[1], jnp.full((pad,), SENT, jnp.int32)]).reshape(E_ROWS, CHK)

    deg_parts = _run_deg(dst2d)
    y3, dinv = _run_mm(x, W, deg_parts)
    acc3 = _run_scatter(y3, src2d, dst2d)
    return _run_epilogue(acc3, y3, dinv, b.reshape(NC, DH))


# speedup vs baseline: 8.8446x; 8.8446x over previous
import functools

import jax
import jax.numpy as jnp
from jax import lax
from jax.experimental import pallas as pl
from jax.experimental.pallas import tpu as pltpu
from jax.experimental.pallas import tpu_sc as plsc

N_NODES = 10000
D_FEAT = 256
DH = 128
N_EDGES = 160000
CHK = 128
E_PAD = 163840
E_ROWS = E_PAD // CHK
NC = 2
NS = 16
LANES = 16
H_PAD = 10240
SENT = N_NODES


def _deg_body(dst2d, deg_parts, hist_sh, idx_v, ones_v, zero_v):
    c = lax.axis_index("c")
    s = lax.axis_index("s")

    def fill(i, _):
        ones_v[i, :] = jnp.full((LANES,), 1.0, jnp.float32)
        zero_v[i, :] = jnp.zeros((LANES,), jnp.float32)
        return 0

    lax.fori_loop(0, CHK, fill, 0)

    stripe = H_PAD // NS
    for k in range(stripe // CHK):
        pltpu.sync_copy(zero_v, hist_sh.at[pl.ds(s * stripe + k * CHK, CHK)])
    plsc.subcore_barrier()

    rows_per_tile = E_ROWS // (NC * NS)
    base = c * (E_ROWS // NC) + s * rows_per_tile
    pltpu.sync_copy(dst2d.at[pl.ds(base, rows_per_tile)], idx_v)
    for j in range(rows_per_tile):
        pltpu.sync_copy(ones_v, hist_sh.at[idx_v.at[j]], add=True)
    plsc.subcore_barrier()

    pltpu.sync_copy(hist_sh.at[pl.ds(s * stripe, stripe)],
                    deg_parts.at[c, pl.ds(s * stripe, stripe)])


def _run_deg(dst2d):
    mesh = plsc.VectorSubcoreMesh(core_axis_name="c", subcore_axis_name="s")
    return pl.kernel(
        _deg_body,
        out_type=jax.ShapeDtypeStruct((NC, H_PAD, LANES), jnp.float32),
        mesh=mesh,
        scratch_types=[
            pltpu.VMEM_SHARED((H_PAD, LANES), jnp.float32),
            pltpu.VMEM((E_ROWS // (NC * NS), CHK), jnp.int32),
            pltpu.VMEM((CHK, LANES), jnp.float32),
            pltpu.VMEM((CHK, LANES), jnp.float32),
        ],
    )(dst2d)


def _scatter_body(y3, src2d, dst2d, acc3, acc_sh, sidx_v, didx_v, rows_v,
                  sem):
    c = lax.axis_index("c")
    s = lax.axis_index("s")

    def fill(i, _):
        for k in range(DH // LANES):
            rows_v[i, pl.ds(k * LANES, LANES)] = jnp.zeros((LANES,),
                                                           jnp.float32)
        return 0

    lax.fori_loop(0, CHK, fill, 0)

    stripe = H_PAD // NS
    for k in range(stripe // CHK):
        pltpu.sync_copy(rows_v, acc_sh.at[pl.ds(s * stripe + k * CHK, CHK)])
    plsc.subcore_barrier()

    rows_per_tile = E_ROWS // NS
    base = s * rows_per_tile
    pltpu.sync_copy(src2d.at[pl.ds(base, rows_per_tile)], sidx_v)
    pltpu.sync_copy(dst2d.at[pl.ds(base, rows_per_tile)], didx_v)

    def step(j, _):
        pltpu.async_copy(y3.at[c].at[sidx_v.at[j]], rows_v, sem).wait()
        pltpu.sync_copy(rows_v, acc_sh.at[didx_v.at[j]], add=True)
        return 0

    lax.fori_loop(0, rows_per_tile, step, 0)
    plsc.subcore_barrier()

    pltpu.sync_copy(acc_sh.at[pl.ds(s * stripe, stripe)],
                    acc3.at[c, pl.ds(s * stripe, stripe)])


def _run_scatter(y3, src2d, dst2d):
    mesh = plsc.VectorSubcoreMesh(core_axis_name="c", subcore_axis_name="s")
    return pl.kernel(
        _scatter_body,
        out_type=jax.ShapeDtypeStruct((NC, H_PAD, DH), jnp.float32),
        mesh=mesh,
        scratch_types=[
            pltpu.VMEM_SHARED((H_PAD, DH), jnp.float32),
            pltpu.VMEM((E_ROWS // NS, CHK), jnp.int32),
            pltpu.VMEM((E_ROWS // NS, CHK), jnp.int32),
            pltpu.VMEM((CHK, DH), jnp.float32),
            pltpu.SemaphoreType.DMA,
        ],
    )(y3, src2d, dst2d)


_RB = 400


def _mm_body(x_ref, w_ref, dp_ref, y3_ref, dinv_ref):
    deg = dp_ref[0, :, 0:1] + dp_ref[1, :, 0:1] + 1.0
    dinv = lax.rsqrt(deg)
    xw = jnp.dot(x_ref[...], w_ref[...], preferred_element_type=jnp.float32)
    y3_ref[0, :, :] = xw * dinv
    dinv_ref[...] = dinv


def _run_mm(x, W, deg_parts):
    grid = (N_NODES // _RB, NC)
    return pl.pallas_call(
        _mm_body,
        grid=grid,
        in_specs=[
            pl.BlockSpec((_RB, D_FEAT), lambda i, c: (i, 0)),
            pl.BlockSpec((D_FEAT, DH), lambda i, c: (0, c)),
            pl.BlockSpec((NC, _RB, LANES), lambda i, c: (0, i, 0)),
        ],
        out_specs=[
            pl.BlockSpec((1, _RB, DH), lambda i, c: (c, i, 0)),
            pl.BlockSpec((_RB, 1), lambda i, c: (i, 0)),
        ],
        out_shape=[
            jax.ShapeDtypeStruct((NC, N_NODES, DH), jnp.float32),
            jax.ShapeDtypeStruct((N_NODES, 1), jnp.float32),
        ],
    )(x, W, deg_parts)


def _ep_body(acc_ref, y_ref, dinv_ref, b_ref, o_ref):
    bh = b_ref[pl.ds(pl.program_id(1), 1), :]
    o_ref[...] = jnp.maximum(
        dinv_ref[...] * (acc_ref[0] + y_ref[0]) + bh, 0.0)


def _run_epilogue(acc3, y3, dinv, b2):
    grid = (N_NODES // _RB, NC)
    return pl.pallas_call(
        _ep_body,
        grid=grid,
        in_specs=[
            pl.BlockSpec((1, _RB, DH), lambda i, c: (c, i, 0)),
            pl.BlockSpec((1, _RB, DH), lambda i, c: (c, i, 0)),
            pl.BlockSpec((_RB, 1), lambda i, c: (i, 0)),
            pl.BlockSpec((NC, DH), lambda i, c: (0, 0)),
        ],
        out_specs=pl.BlockSpec((_RB, DH), lambda i, c: (i, c)),
        out_shape=jax.ShapeDtypeStruct((N_NODES, D_FEAT), jnp.float32),
    )(acc3, y3, dinv, b2)


@jax.jit
def kernel(x, edge_index, W, b):
    ei = edge_index.astype(jnp.int32)
    pad = E_PAD - N_EDGES
    src2d = jnp.concatenate(
        [ei[0], jnp.zeros((pad,), jnp.int32)]).reshape(E_ROWS, CHK)
    dst2d = jnp.concatenate(
        [ei[1], jnp.full((pad,), SENT, jnp.int32)]).reshape(E_ROWS, CHK)

    deg_parts = _run_deg(dst2d)
    y3, dinv = _run_mm(x, W, deg_parts)
    acc3 = _run_scatter(y3, src2d, dst2d)
    return _run_epilogue(acc3, y3, dinv, b.reshape(NC, DH))


# speedup vs baseline: 10.1405x; 1.1465x over previous
import functools

import jax
import jax.numpy as jnp
from jax import lax
from jax.experimental import pallas as pl
from jax.experimental.pallas import tpu as pltpu
from jax.experimental.pallas import tpu_sc as plsc

N_NODES = 10000
D_FEAT = 256
DH = 128
N_EDGES = 160000
CHK = 128
E_PAD = 163840
E_ROWS = E_PAD // CHK
NC = 2
NS = 16
LANES = 16
H_PAD = 10240
SENT = N_NODES


def _deg_body(dst2d, deg_parts, hist_sh, idx_v, ones_v, zero_v):
    c = lax.axis_index("c")
    s = lax.axis_index("s")

    def fill(i, _):
        ones_v[i, :] = jnp.full((LANES,), 1.0, jnp.float32)
        zero_v[i, :] = jnp.zeros((LANES,), jnp.float32)
        return 0

    lax.fori_loop(0, CHK, fill, 0)

    stripe = H_PAD // NS
    for k in range(stripe // CHK):
        pltpu.sync_copy(zero_v, hist_sh.at[pl.ds(s * stripe + k * CHK, CHK)])
    plsc.subcore_barrier()

    rows_per_tile = E_ROWS // (NC * NS)
    base = c * (E_ROWS // NC) + s * rows_per_tile
    pltpu.sync_copy(dst2d.at[pl.ds(base, rows_per_tile)], idx_v)
    for j in range(rows_per_tile):
        pltpu.sync_copy(ones_v, hist_sh.at[idx_v.at[j]], add=True)
    plsc.subcore_barrier()

    pltpu.sync_copy(hist_sh.at[pl.ds(s * stripe, stripe)],
                    deg_parts.at[c, pl.ds(s * stripe, stripe)])


def _run_deg(dst2d):
    mesh = plsc.VectorSubcoreMesh(core_axis_name="c", subcore_axis_name="s")
    return pl.kernel(
        _deg_body,
        out_type=jax.ShapeDtypeStruct((NC, H_PAD, LANES), jnp.float32),
        mesh=mesh,
        scratch_types=[
            pltpu.VMEM_SHARED((H_PAD, LANES), jnp.float32),
            pltpu.VMEM((E_ROWS // (NC * NS), CHK), jnp.int32),
            pltpu.VMEM((CHK, LANES), jnp.float32),
            pltpu.VMEM((CHK, LANES), jnp.float32),
        ],
    )(dst2d)


def _scatter_body(y3, src2d, dst2d, acc3, acc_sh, sidx_v, didx_v, rows0_v,
                  rows1_v, sem0, sem1):
    c = lax.axis_index("c")
    s = lax.axis_index("s")

    def fill(i, _):
        for k in range(DH // LANES):
            rows0_v[i, pl.ds(k * LANES, LANES)] = jnp.zeros((LANES,),
                                                            jnp.float32)
        return 0

    lax.fori_loop(0, CHK, fill, 0)

    stripe = H_PAD // NS
    for k in range(stripe // CHK):
        pltpu.sync_copy(rows0_v, acc_sh.at[pl.ds(s * stripe + k * CHK, CHK)])
    plsc.subcore_barrier()

    rows_per_tile = E_ROWS // NS
    HALF = rows_per_tile // 2

    def gather(j, buf, sem):
        return pltpu.async_copy(y3.at[c].at[sidx_v.at[j]], buf, sem)

    def scat(j, buf):
        pltpu.sync_copy(buf, acc_sh.at[didx_v.at[j]], add=True)

    for half in range(2):
        base = s * rows_per_tile + half * HALF
        pltpu.sync_copy(src2d.at[pl.ds(base, HALF)], sidx_v)
        pltpu.sync_copy(dst2d.at[pl.ds(base, HALF)], didx_v)
        gather(0, rows0_v, sem0)

        def pair(g, _):
            j0 = 2 * g
            gather(j0 + 1, rows1_v, sem1)
            pltpu.make_async_copy(y3.at[c, pl.ds(0, CHK)], rows0_v,
                                  sem0).wait()
            scat(j0, rows0_v)

            @pl.when(g < HALF // 2 - 1)
            def _():
                gather(j0 + 2, rows0_v, sem0)

            pltpu.make_async_copy(y3.at[c, pl.ds(0, CHK)], rows1_v,
                                  sem1).wait()
            scat(j0 + 1, rows1_v)
            return 0

        lax.fori_loop(0, HALF // 2, pair, 0)

    plsc.subcore_barrier()

    pltpu.sync_copy(acc_sh.at[pl.ds(s * stripe, stripe)],
                    acc3.at[c, pl.ds(s * stripe, stripe)])


def _run_scatter(y3, src2d, dst2d):
    mesh = plsc.VectorSubcoreMesh(core_axis_name="c", subcore_axis_name="s")
    return pl.kernel(
        _scatter_body,
        out_type=jax.ShapeDtypeStruct((NC, H_PAD, DH), jnp.float32),
        mesh=mesh,
        scratch_types=[
            pltpu.VMEM_SHARED((H_PAD, DH), jnp.float32),
            pltpu.VMEM((E_ROWS // NS // 2, CHK), jnp.int32),
            pltpu.VMEM((E_ROWS // NS // 2, CHK), jnp.int32),
            pltpu.VMEM((CHK, DH), jnp.float32),
            pltpu.VMEM((CHK, DH), jnp.float32),
            pltpu.SemaphoreType.DMA,
            pltpu.SemaphoreType.DMA,
        ],
    )(y3, src2d, dst2d)


_RB = 400


def _mm_body(x_ref, w_ref, dp_ref, y3_ref, dinv_ref):
    deg = dp_ref[0, :, 0:1] + dp_ref[1, :, 0:1] + 1.0
    dinv = lax.rsqrt(deg)
    xw = jnp.dot(x_ref[...], w_ref[...], preferred_element_type=jnp.float32)
    y3_ref[0, :, :] = xw * dinv
    dinv_ref[...] = dinv


def _run_mm(x, W, deg_parts):
    grid = (N_NODES // _RB, NC)
    return pl.pallas_call(
        _mm_body,
        grid=grid,
        in_specs=[
            pl.BlockSpec((_RB, D_FEAT), lambda i, c: (i, 0)),
            pl.BlockSpec((D_FEAT, DH), lambda i, c: (0, c)),
            pl.BlockSpec((NC, _RB, LANES), lambda i, c: (0, i, 0)),
        ],
        out_specs=[
            pl.BlockSpec((1, _RB, DH), lambda i, c: (c, i, 0)),
            pl.BlockSpec((_RB, 1), lambda i, c: (i, 0)),
        ],
        out_shape=[
            jax.ShapeDtypeStruct((NC, N_NODES, DH), jnp.float32),
            jax.ShapeDtypeStruct((N_NODES, 1), jnp.float32),
        ],
    )(x, W, deg_parts)


def _ep_body(acc_ref, y_ref, dinv_ref, b_ref, o_ref):
    bh = b_ref[pl.ds(pl.program_id(1), 1), :]
    o_ref[...] = jnp.maximum(
        dinv_ref[...] * (acc_ref[0] + y_ref[0]) + bh, 0.0)


def _run_epilogue(acc3, y3, dinv, b2):
    grid = (N_NODES // _RB, NC)
    return pl.pallas_call(
        _ep_body,
        grid=grid,
        in_specs=[
            pl.BlockSpec((1, _RB, DH), lambda i, c: (c, i, 0)),
            pl.BlockSpec((1, _RB, DH), lambda i, c: (c, i, 0)),
            pl.BlockSpec((_RB, 1), lambda i, c: (i, 0)),
            pl.BlockSpec((NC, DH), lambda i, c: (0, 0)),
        ],
        out_specs=pl.BlockSpec((_RB, DH), lambda i, c: (i, c)),
        out_shape=jax.ShapeDtypeStruct((N_NODES, D_FEAT), jnp.float32),
    )(acc3, y3, dinv, b2)


@jax.jit
def kernel(x, edge_index, W, b):
    ei = edge_index.astype(jnp.int32)
    pad = E_PAD - N_EDGES
    src2d = jnp.concatenate(
        [ei[0], jnp.zeros((pad,), jnp.int32)]).reshape(E_ROWS, CHK)
    dst2d = jnp.concatenate(
        [ei[1], jnp.full((pad,), SENT, jnp.int32)]).reshape(E_ROWS, CHK)

    deg_parts = _run_deg(dst2d)
    y3, dinv = _run_mm(x, W, deg_parts)
    acc3 = _run_scatter(y3, src2d, dst2d)
    return _run_epilogue(acc3, y3, dinv, b.reshape(NC, DH))
